# baseline (speedup 1.0000x reference)
import functools

import jax
import jax.numpy as jnp
from jax import lax
from jax.experimental import pallas as pl
from jax.experimental.pallas import tpu as pltpu

_MASK_VAL = -1000000000000.0

_DOT_T = (((1,), (1,)), ((), ()))


def _head_kernel(feats_ref, wv_ref, wn_ref, mb_ref, ov_ref, on_ref):
    num_verbs = ov_ref.shape[0]
    x = feats_ref[...].astype(jnp.bfloat16)
    mbv = mb_ref[:num_verbs, :]
    v = lax.dot_general(wv_ref[...].astype(jnp.bfloat16), x, _DOT_T,
                        preferred_element_type=jnp.float32)
    ov_ref[...] = v * mbv[:, 0:1] + mbv[:, 1:2]
    mbn = mb_ref[num_verbs:, :]
    n = lax.dot_general(wn_ref[...].astype(jnp.bfloat16), x, _DOT_T,
                        preferred_element_type=jnp.float32)
    on_ref[...] = n * mbn[:, 0:1] + mbn[:, 1:2]


@functools.partial(jax.jit, static_argnames=("block_m",))
def _masked_head(feats, W_verb, b_verb, W_noun, b_noun,
                 seen_verb_mask, seen_noun_mask, block_m=2048):
    batch, d_feat = feats.shape
    num_verbs = W_verb.shape[1]
    num_nouns = W_noun.shape[1]
    grid = (batch // block_m,)

    wv = W_verb.T
    wn = W_noun.T
    mask = jnp.concatenate([seen_verb_mask, seen_noun_mask])
    bias = jnp.concatenate([b_verb, b_noun])
    mask_f = mask.astype(jnp.float32)
    bias_or_mask = jnp.where(mask, bias, _MASK_VAL)
    mb = jnp.stack([mask_f, bias_or_mask], axis=1)

    full = lambda *shape: pl.BlockSpec(shape, lambda i: (0,) * len(shape))
    vt, nt = pl.pallas_call(
        _head_kernel,
        grid=grid,
        in_specs=[
            pl.BlockSpec((block_m, d_feat), lambda i: (i, 0)),
            full(num_verbs, d_feat),
            full(num_nouns, d_feat),
            full(num_verbs + num_nouns, 2),
        ],
        out_specs=(
            pl.BlockSpec((num_verbs, block_m), lambda i: (0, i)),
            pl.BlockSpec((num_nouns, block_m), lambda i: (0, i)),
        ),
        out_shape=(
            jax.ShapeDtypeStruct((num_verbs, batch), jnp.float32),
            jax.ShapeDtypeStruct((num_nouns, batch), jnp.float32),
        ),
        compiler_params=pltpu.CompilerParams(
            dimension_semantics=("parallel",),
        ),
    )(feats, wv, wn, mb)
    return vt.T, nt.T


def kernel(feats, W_verb, b_verb, W_noun, b_noun, seen_verb_mask, seen_noun_mask):
    return _masked_head(feats, W_verb, b_verb, W_noun, b_noun,
                        seen_verb_mask, seen_noun_mask)


# speedup vs baseline: 1.0168x; 1.0168x over previous
import functools

import jax
import jax.numpy as jnp
from jax import lax
from jax.experimental import pallas as pl
from jax.experimental.pallas import tpu as pltpu

_MASK_VAL = -1000000000000.0

_DOT_T = (((1,), (1,)), ((), ()))


def _head_kernel(feats_ref, wv_ref, wn_ref, mb_ref, ov_ref, on_ref):
    num_verbs = ov_ref.shape[0]
    x = feats_ref[...].astype(jnp.bfloat16)
    mb = mb_ref[...].T
    mbv = mb[:num_verbs, :]
    v = lax.dot_general(wv_ref[...].astype(jnp.bfloat16), x, _DOT_T,
                        preferred_element_type=jnp.float32)
    ov_ref[...] = v * mbv[:, 0:1] + mbv[:, 1:2]
    mbn = mb[num_verbs:, :]
    n = lax.dot_general(wn_ref[...].astype(jnp.bfloat16), x, _DOT_T,
                        preferred_element_type=jnp.float32)
    on_ref[...] = n * mbn[:, 0:1] + mbn[:, 1:2]


@functools.partial(jax.jit, static_argnames=("block_m",))
def _masked_head(feats, W_verb, b_verb, W_noun, b_noun,
                 seen_verb_mask, seen_noun_mask, block_m=4096):
    batch, d_feat = feats.shape
    num_verbs = W_verb.shape[1]
    num_nouns = W_noun.shape[1]
    grid = (batch // block_m,)

    wv = W_verb.T
    wn = W_noun.T
    mask = jnp.concatenate([seen_verb_mask, seen_noun_mask])
    bias = jnp.concatenate([b_verb, b_noun])
    mask_f = mask.astype(jnp.float32)
    bias_or_mask = jnp.where(mask, bias, _MASK_VAL)
    mb = jnp.stack([mask_f, bias_or_mask], axis=0)

    full = lambda *shape: pl.BlockSpec(shape, lambda i: (0,) * len(shape))
    vt, nt = pl.pallas_call(
        _head_kernel,
        grid=grid,
        in_specs=[
            pl.BlockSpec((block_m, d_feat), lambda i: (i, 0)),
            full(num_verbs, d_feat),
            full(num_nouns, d_feat),
            full(2, num_verbs + num_nouns),
        ],
        out_specs=(
            pl.BlockSpec((num_verbs, block_m), lambda i: (0, i)),
            pl.BlockSpec((num_nouns, block_m), lambda i: (0, i)),
        ),
        out_shape=(
            jax.ShapeDtypeStruct((num_verbs, batch), jnp.float32),
            jax.ShapeDtypeStruct((num_nouns, batch), jnp.float32),
        ),
        compiler_params=pltpu.CompilerParams(
            dimension_semantics=("parallel",),
        ),
    )(feats, wv, wn, mb)
    return vt.T, nt.T


def kernel(feats, W_verb, b_verb, W_noun, b_noun, seen_verb_mask, seen_noun_mask):
    return _masked_head(feats, W_verb, b_verb, W_noun, b_noun,
                        seen_verb_mask, seen_noun_mask)


# speedup vs baseline: 1.0265x; 1.0095x over previous
import functools

import jax
import jax.numpy as jnp
from jax import lax
from jax.experimental import pallas as pl
from jax.experimental.pallas import tpu as pltpu

_MASK_VAL = -1000000000000.0

_DOT_T = (((1,), (1,)), ((), ()))


def _head_kernel(feats_ref, wv_ref, wn_ref, bv_ref, bn_ref, mv_ref, mn_ref,
                 ov_ref, on_ref):
    x = feats_ref[...].astype(jnp.bfloat16)
    mv = mv_ref[...].astype(jnp.float32).T
    bv = jnp.where(mv != 0.0, bv_ref[...].T, _MASK_VAL)
    v = lax.dot_general(wv_ref[...].astype(jnp.bfloat16), x, _DOT_T,
                        preferred_element_type=jnp.float32)
    ov_ref[...] = v * mv + bv
    mn = mn_ref[...].astype(jnp.float32).T
    bn = jnp.where(mn != 0.0, bn_ref[...].T, _MASK_VAL)
    n = lax.dot_general(wn_ref[...].astype(jnp.bfloat16), x, _DOT_T,
                        preferred_element_type=jnp.float32)
    on_ref[...] = n * mn + bn


@functools.partial(jax.jit, static_argnames=("block_m",))
def _masked_head(feats, W_verb, b_verb, W_noun, b_noun,
                 seen_verb_mask, seen_noun_mask, block_m=4096):
    batch, d_feat = feats.shape
    num_verbs = W_verb.shape[1]
    num_nouns = W_noun.shape[1]
    grid = (batch // block_m,)

    wv = W_verb.T
    wn = W_noun.T
    bv = b_verb.reshape(1, num_verbs)
    bn = b_noun.reshape(1, num_nouns)
    mv = seen_verb_mask.reshape(1, num_verbs)
    mn = seen_noun_mask.reshape(1, num_nouns)

    full = lambda *shape: pl.BlockSpec(shape, lambda i: (0,) * len(shape))
    vt, nt = pl.pallas_call(
        _head_kernel,
        grid=grid,
        in_specs=[
            pl.BlockSpec((block_m, d_feat), lambda i: (i, 0)),
            full(num_verbs, d_feat),
            full(num_nouns, d_feat),
            full(1, num_verbs),
            full(1, num_nouns),
            full(1, num_verbs),
            full(1, num_nouns),
        ],
        out_specs=(
            pl.BlockSpec((num_verbs, block_m), lambda i: (0, i)),
            pl.BlockSpec((num_nouns, block_m), lambda i: (0, i)),
        ),
        out_shape=(
            jax.ShapeDtypeStruct((num_verbs, batch), jnp.float32),
            jax.ShapeDtypeStruct((num_nouns, batch), jnp.float32),
        ),
        compiler_params=pltpu.CompilerParams(
            dimension_semantics=("parallel",),
        ),
    )(feats, wv, wn, bv, bn, mv, mn)
    return vt.T, nt.T


def kernel(feats, W_verb, b_verb, W_noun, b_noun, seen_verb_mask, seen_noun_mask):
    return _masked_head(feats, W_verb, b_verb, W_noun, b_noun,
                        seen_verb_mask, seen_noun_mask)


# speedup vs baseline: 1.0294x; 1.0029x over previous
import functools

import jax
import jax.numpy as jnp
from jax import lax
from jax.experimental import pallas as pl
from jax.experimental.pallas import tpu as pltpu

_MASK_VAL = -1000000000000.0

_DOT_T = (((1,), (1,)), ((), ()))


def _head_kernel(feats_ref, wv_ref, wn_ref, bv_ref, bn_ref, mv_ref, mn_ref,
                 ov_ref, on_ref):
    x = feats_ref[...].astype(jnp.bfloat16)
    mv = mv_ref[...].reshape(1, -1).astype(jnp.float32).T
    bv = jnp.where(mv != 0.0, bv_ref[...].T, _MASK_VAL)
    v = lax.dot_general(wv_ref[...].astype(jnp.bfloat16), x, _DOT_T,
                        preferred_element_type=jnp.float32)
    ov_ref[...] = v * mv + bv
    mn = mn_ref[...].reshape(1, -1).astype(jnp.float32).T
    bn = jnp.where(mn != 0.0, bn_ref[...].T, _MASK_VAL)
    n = lax.dot_general(wn_ref[...].astype(jnp.bfloat16), x, _DOT_T,
                        preferred_element_type=jnp.float32)
    on_ref[...] = n * mn + bn


@functools.partial(jax.jit, static_argnames=("block_m",))
def _masked_head(feats, W_verb, b_verb, W_noun, b_noun,
                 seen_verb_mask, seen_noun_mask, block_m=4096):
    batch, d_feat = feats.shape
    num_verbs = W_verb.shape[1]
    num_nouns = W_noun.shape[1]
    grid = (batch // block_m,)

    wv = W_verb.T
    wn = W_noun.T
    bv = b_verb.reshape(1, num_verbs)
    bn = b_noun.reshape(1, num_nouns)
    mv = seen_verb_mask
    mn = seen_noun_mask

    full = lambda *shape: pl.BlockSpec(shape, lambda i: (0,) * len(shape))
    vt, nt = pl.pallas_call(
        _head_kernel,
        grid=grid,
        in_specs=[
            pl.BlockSpec((block_m, d_feat), lambda i: (i, 0)),
            full(num_verbs, d_feat),
            full(num_nouns, d_feat),
            full(1, num_verbs),
            full(1, num_nouns),
            full(num_verbs),
            full(num_nouns),
        ],
        out_specs=(
            pl.BlockSpec((num_verbs, block_m), lambda i: (0, i)),
            pl.BlockSpec((num_nouns, block_m), lambda i: (0, i)),
        ),
        out_shape=(
            jax.ShapeDtypeStruct((num_verbs, batch), jnp.float32),
            jax.ShapeDtypeStruct((num_nouns, batch), jnp.float32),
        ),
        compiler_params=pltpu.CompilerParams(
            dimension_semantics=("parallel",),
        ),
    )(feats, wv, wn, bv, bn, mv, mn)
    return vt.T, nt.T


def kernel(feats, W_verb, b_verb, W_noun, b_noun, seen_verb_mask, seen_noun_mask):
    return _masked_head(feats, W_verb, b_verb, W_noun, b_noun,
                        seen_verb_mask, seen_noun_mask)


# speedup vs baseline: 1.0562x; 1.0260x over previous
import functools

import jax
import jax.numpy as jnp
from jax import lax
from jax.experimental import pallas as pl
from jax.experimental.pallas import tpu as pltpu

_MASK_VAL = -1000000000000.0

_DOT_T = (((1,), (1,)), ((), ()))


def _head_kernel(feats_ref, wv_ref, wn_ref, bv_ref, bn_ref, mm_ref,
                 ov_ref, on_ref):
    num_verbs = ov_ref.shape[0]
    x = feats_ref[...].astype(jnp.bfloat16)
    m = mm_ref[...].reshape(1, -1).T
    mv = m[:num_verbs, :]
    bv = jnp.where(mv != 0.0, bv_ref[...].T, _MASK_VAL)
    v = lax.dot_general(wv_ref[...].astype(jnp.bfloat16), x, _DOT_T,
                        preferred_element_type=jnp.float32)
    ov_ref[...] = v * mv + bv
    mn = m[num_verbs:, :]
    bn = jnp.where(mn != 0.0, bn_ref[...].T, _MASK_VAL)
    n = lax.dot_general(wn_ref[...].astype(jnp.bfloat16), x, _DOT_T,
                        preferred_element_type=jnp.float32)
    on_ref[...] = n * mn + bn


@functools.partial(jax.jit, static_argnames=("block_m",))
def _masked_head(feats, W_verb, b_verb, W_noun, b_noun,
                 seen_verb_mask, seen_noun_mask, block_m=4096):
    batch, d_feat = feats.shape
    num_verbs = W_verb.shape[1]
    num_nouns = W_noun.shape[1]
    grid = (batch // block_m,)

    wv = W_verb.T
    wn = W_noun.T
    bv = b_verb.reshape(1, num_verbs)
    bn = b_noun.reshape(1, num_nouns)
    mm = jnp.concatenate([seen_verb_mask, seen_noun_mask]).astype(jnp.float32)

    full = lambda *shape: pl.BlockSpec(shape, lambda i: (0,) * len(shape))
    vt, nt = pl.pallas_call(
        _head_kernel,
        grid=grid,
        in_specs=[
            pl.BlockSpec((block_m, d_feat), lambda i: (i, 0)),
            full(num_verbs, d_feat),
            full(num_nouns, d_feat),
            full(1, num_verbs),
            full(1, num_nouns),
            full(num_verbs + num_nouns),
        ],
        out_specs=(
            pl.BlockSpec((num_verbs, block_m), lambda i: (0, i)),
            pl.BlockSpec((num_nouns, block_m), lambda i: (0, i)),
        ),
        out_shape=(
            jax.ShapeDtypeStruct((num_verbs, batch), jnp.float32),
            jax.ShapeDtypeStruct((num_nouns, batch), jnp.float32),
        ),
        compiler_params=pltpu.CompilerParams(
            dimension_semantics=("parallel",),
        ),
    )(feats, wv, wn, bv, bn, mm)
    return vt.T, nt.T


def kernel(feats, W_verb, b_verb, W_noun, b_noun, seen_verb_mask, seen_noun_mask):
    return _masked_head(feats, W_verb, b_verb, W_noun, b_noun,
                        seen_verb_mask, seen_noun_mask)


# speedup vs baseline: 1.0995x; 1.0409x over previous
import functools

import jax
import jax.numpy as jnp
from jax import lax
from jax.experimental import pallas as pl
from jax.experimental.pallas import tpu as pltpu

_MASK_VAL = -1000000000000.0

_DOT_T = (((1,), (1,)), ((), ()))


def _head_kernel(feats_ref, wv_ref, wn_ref, bv_ref, bn_ref, mv_ref, mn_ref,
                 ov_ref, on_ref):
    x = feats_ref[...].astype(jnp.bfloat16)
    mn = mn_ref[...].reshape(1, -1).T.astype(jnp.float32)
    bn = jnp.where(mn != 0.0, bn_ref[...].T, _MASK_VAL)
    n = lax.dot_general(wn_ref[...].astype(jnp.bfloat16), x, _DOT_T,
                        preferred_element_type=jnp.float32)
    on_ref[...] = n * mn + bn
    mv = mv_ref[...].reshape(1, -1).T.astype(jnp.float32)
    bv = jnp.where(mv != 0.0, bv_ref[...].T, _MASK_VAL)
    v = lax.dot_general(wv_ref[...].astype(jnp.bfloat16), x, _DOT_T,
                        preferred_element_type=jnp.float32)
    ov_ref[...] = v * mv + bv


@functools.partial(jax.jit, static_argnames=("block_m",))
def _masked_head(feats, W_verb, b_verb, W_noun, b_noun,
                 seen_verb_mask, seen_noun_mask, block_m=4096):
    batch, d_feat = feats.shape
    num_verbs = W_verb.shape[1]
    num_nouns = W_noun.shape[1]
    grid = (batch // block_m,)

    wv = W_verb.T
    wn = W_noun.T
    bv = b_verb.reshape(1, num_verbs)
    bn = b_noun.reshape(1, num_nouns)
    mv = seen_verb_mask.view(jnp.int8)
    mn = seen_noun_mask.view(jnp.int8)

    full = lambda *shape: pl.BlockSpec(shape, lambda i: (0,) * len(shape))
    vt, nt = pl.pallas_call(
        _head_kernel,
        grid=grid,
        in_specs=[
            pl.BlockSpec((block_m, d_feat), lambda i: (i, 0)),
            full(num_verbs, d_feat),
            full(num_nouns, d_feat),
            full(1, num_verbs),
            full(1, num_nouns),
            full(num_verbs),
            full(num_nouns),
        ],
        out_specs=(
            pl.BlockSpec((num_verbs, block_m), lambda i: (0, i)),
            pl.BlockSpec((num_nouns, block_m), lambda i: (0, i)),
        ),
        out_shape=(
            jax.ShapeDtypeStruct((num_verbs, batch), jnp.float32),
            jax.ShapeDtypeStruct((num_nouns, batch), jnp.float32),
        ),
        compiler_params=pltpu.CompilerParams(
            dimension_semantics=("parallel",),
        ),
    )(feats, wv, wn, bv, bn, mv, mn)
    return vt.T, nt.T


def kernel(feats, W_verb, b_verb, W_noun, b_noun, seen_verb_mask, seen_noun_mask):
    return _masked_head(feats, W_verb, b_verb, W_noun, b_noun,
                        seen_verb_mask, seen_noun_mask)


# speedup vs baseline: 1.1114x; 1.0109x over previous
import functools

import jax
import jax.numpy as jnp
from jax import lax
from jax.experimental import pallas as pl
from jax.experimental.pallas import tpu as pltpu

_MASK_VAL = -1000000000000.0

_DOT_T = (((1,), (1,)), ((), ()))


def _head_kernel(feats_ref, wv_ref, wn_ref, bv_ref, bn_ref, mm_ref,
                 ov_ref, on_ref):
    num_verbs = ov_ref.shape[0]
    x = feats_ref[...].astype(jnp.bfloat16)
    m = mm_ref[...].reshape(1, -1).T.astype(jnp.float32)
    mn = m[num_verbs:, :]
    bn = jnp.where(mn != 0.0, bn_ref[...].T, _MASK_VAL)
    n = lax.dot_general(wn_ref[...].astype(jnp.bfloat16), x, _DOT_T,
                        preferred_element_type=jnp.float32)
    on_ref[...] = n * mn + bn
    mv = m[:num_verbs, :]
    bv = jnp.where(mv != 0.0, bv_ref[...].T, _MASK_VAL)
    v = lax.dot_general(wv_ref[...].astype(jnp.bfloat16), x, _DOT_T,
                        preferred_element_type=jnp.float32)
    ov_ref[...] = v * mv + bv


@functools.partial(jax.jit, static_argnames=("block_m",))
def _masked_head(feats, W_verb, b_verb, W_noun, b_noun,
                 seen_verb_mask, seen_noun_mask, block_m=4096):
    batch, d_feat = feats.shape
    num_verbs = W_verb.shape[1]
    num_nouns = W_noun.shape[1]
    grid = (batch // block_m,)

    wv = W_verb.T
    wn = W_noun.T
    bv = b_verb.reshape(1, num_verbs)
    bn = b_noun.reshape(1, num_nouns)
    mm = jnp.concatenate([seen_verb_mask.view(jnp.int8),
                          seen_noun_mask.view(jnp.int8)])

    full = lambda *shape: pl.BlockSpec(shape, lambda i: (0,) * len(shape))
    vt, nt = pl.pallas_call(
        _head_kernel,
        grid=grid,
        in_specs=[
            pl.BlockSpec((block_m, d_feat), lambda i: (i, 0)),
            full(num_verbs, d_feat),
            full(num_nouns, d_feat),
            full(1, num_verbs),
            full(1, num_nouns),
            full(num_verbs + num_nouns),
        ],
        out_specs=(
            pl.BlockSpec((num_verbs, block_m), lambda i: (0, i)),
            pl.BlockSpec((num_nouns, block_m), lambda i: (0, i)),
        ),
        out_shape=(
            jax.ShapeDtypeStruct((num_verbs, batch), jnp.float32),
            jax.ShapeDtypeStruct((num_nouns, batch), jnp.float32),
        ),
        compiler_params=pltpu.CompilerParams(
            dimension_semantics=("parallel",),
        ),
    )(feats, wv, wn, bv, bn, mm)
    return vt.T, nt.T


def kernel(feats, W_verb, b_verb, W_noun, b_noun, seen_verb_mask, seen_noun_mask):
    return _masked_head(feats, W_verb, b_verb, W_noun, b_noun,
                        seen_verb_mask, seen_noun_mask)


# speedup vs baseline: 1.1329x; 1.0193x over previous
import functools

import jax
import jax.numpy as jnp
from jax import lax
from jax.experimental import pallas as pl
from jax.experimental.pallas import tpu as pltpu

_MASK_VAL = -1000000000000.0

_DOT_T = (((1,), (1,)), ((), ()))


def _head_kernel(feats_ref, wv_ref, wn_ref, bv_ref, bn_ref, mm_ref,
                 ov_ref, on_ref):
    num_verbs = ov_ref.shape[0]
    x = feats_ref[...].astype(jnp.bfloat16)
    m = mm_ref[...].reshape(1, -1).T
    mn = m[num_verbs:, :]
    bn = jnp.where(mn != 0.0, bn_ref[...].T, _MASK_VAL)
    n = lax.dot_general(wn_ref[...].astype(jnp.bfloat16), x, _DOT_T,
                        preferred_element_type=jnp.float32)
    on_ref[...] = n * mn + bn
    mv = m[:num_verbs, :]
    bv = jnp.where(mv != 0.0, bv_ref[...].T, _MASK_VAL)
    v = lax.dot_general(wv_ref[...].astype(jnp.bfloat16), x, _DOT_T,
                        preferred_element_type=jnp.float32)
    ov_ref[...] = v * mv + bv


@functools.partial(jax.jit, static_argnames=("block_m",))
def _masked_head(feats, W_verb, b_verb, W_noun, b_noun,
                 seen_verb_mask, seen_noun_mask, block_m=4096):
    batch, d_feat = feats.shape
    num_verbs = W_verb.shape[1]
    num_nouns = W_noun.shape[1]
    grid = (batch // block_m,)

    wv = W_verb.T
    wn = W_noun.T
    bv = b_verb.reshape(1, num_verbs)
    bn = b_noun.reshape(1, num_nouns)
    mm = jnp.concatenate([seen_verb_mask, seen_noun_mask]).astype(jnp.float32)

    full = lambda *shape: pl.BlockSpec(shape, lambda i: (0,) * len(shape))
    vt, nt = pl.pallas_call(
        _head_kernel,
        grid=grid,
        in_specs=[
            pl.BlockSpec((block_m, d_feat), lambda i: (i, 0)),
            full(num_verbs, d_feat),
            full(num_nouns, d_feat),
            full(1, num_verbs),
            full(1, num_nouns),
            full(num_verbs + num_nouns),
        ],
        out_specs=(
            pl.BlockSpec((num_verbs, block_m), lambda i: (0, i)),
            pl.BlockSpec((num_nouns, block_m), lambda i: (0, i)),
        ),
        out_shape=(
            jax.ShapeDtypeStruct((num_verbs, batch), jnp.float32),
            jax.ShapeDtypeStruct((num_nouns, batch), jnp.float32),
        ),
        compiler_params=pltpu.CompilerParams(
            dimension_semantics=("parallel",),
        ),
    )(feats, wv, wn, bv, bn, mm)
    return vt.T, nt.T


def kernel(feats, W_verb, b_verb, W_noun, b_noun, seen_verb_mask, seen_noun_mask):
    return _masked_head(feats, W_verb, b_verb, W_noun, b_noun,
                        seen_verb_mask, seen_noun_mask)
